# baseline (speedup 1.0000x reference)
import functools

import jax
import jax.numpy as jnp
from jax import lax
from jax.experimental import pallas as pl
from jax.experimental.pallas import tpu as pltpu
from jax.experimental.pallas import tpu_sc as plsc

NC = 2
NS = 16
NW = NC * NS
B = 24576
BPW = B // NW


def _make_gather(Dp: int, R: int):
    nchunks = BPW // R
    mesh = plsc.VectorSubcoreMesh(core_axis_name="c", subcore_axis_name="s")

    @functools.partial(
        pl.kernel,
        out_type=jax.ShapeDtypeStruct((B, Dp), jnp.float32),
        mesh=mesh,
        scratch_types=[
            pltpu.VMEM((R,), jnp.int32),
            pltpu.VMEM((R, Dp), jnp.float32),
            pltpu.SemaphoreType.DMA,
        ],
        compiler_params=pltpu.CompilerParams(use_tc_tiling_on_sc=False),
    )
    def k(idx_hbm, table_hbm, out_hbm, idx_c, rows_v, sem):
        wid = lax.axis_index("s") * NC + lax.axis_index("c")
        base = wid * BPW

        def body(c, carry):
            off = base + pl.multiple_of(c * R, 8)
            pltpu.sync_copy(idx_hbm.at[pl.ds(off, R)], idx_c)
            pltpu.async_copy(table_hbm.at[idx_c], rows_v, sem).wait()
            pltpu.sync_copy(rows_v, out_hbm.at[pl.ds(off, R)])
            return carry

        lax.fori_loop(0, nchunks, body, 0)

    return k


def _pad8(d: int) -> int:
    return (d + 7) // 8 * 8


def _make_depad(D: int, Dp: int, GI: int):
    def body(in_ref, out_ref):
        out_ref[...] = in_ref[...][:, :D].reshape(GI, 6, D)

    return pl.pallas_call(
        body,
        grid=(4096 // GI,),
        in_specs=[pl.BlockSpec((GI * 6, Dp), lambda i: (i, 0))],
        out_specs=pl.BlockSpec((GI, 6, D), lambda i: (i, 0, 0)),
        out_shape=jax.ShapeDtypeStruct((4096, 6, D), jnp.float32),
    )


_CHUNK = {2047: (24, 16), 1023: (48, 32), 511: (96, 64), 19: (128, 128)}
_KERNELS = {
    D: (_make_gather(_pad8(D), R), _make_depad(D, _pad8(D), GI))
    for D, (R, GI) in _CHUNK.items()
}


def kernel(x, species, abilities, items, movesets, teratypes):
    idx = x.reshape(-1).astype(jnp.int32)
    outs = []
    for table in (species, abilities, items, movesets, teratypes):
        D = table.shape[1]
        Dp = _pad8(D)
        tp = table if Dp == D else jnp.pad(table, ((0, 0), (0, Dp - D)))
        gather, depad = _KERNELS[D]
        outs.append(depad(gather(idx, tp)))
    return tuple(outs)


# speedup vs baseline: 1.4364x; 1.4364x over previous
import functools

import jax
import jax.numpy as jnp
from jax import lax
from jax.experimental import pallas as pl
from jax.experimental.pallas import tpu as pltpu
from jax.experimental.pallas import tpu_sc as plsc

NC = 2
NS = 16
NW = NC * NS
B = 24576
BPW = B // NW


def _make_gather_grouped(Dp: int, R: int):
    nchunks = BPW // R
    ngroups = R // 6
    mesh = plsc.VectorSubcoreMesh(core_axis_name="c", subcore_axis_name="s")

    @functools.partial(
        pl.kernel,
        out_type=jax.ShapeDtypeStruct((4096 * 8, Dp), jnp.float32),
        mesh=mesh,
        scratch_types=[
            pltpu.VMEM((R,), jnp.int32),
            pltpu.VMEM((R, Dp), jnp.float32),
            pltpu.SemaphoreType.DMA,
        ],
        compiler_params=pltpu.CompilerParams(use_tc_tiling_on_sc=False),
    )
    def k(idx_hbm, table_hbm, out_hbm, idx_c, rows_v, sem):
        wid = lax.axis_index("s") * NC + lax.axis_index("c")
        base = wid * BPW

        def body(c, carry):
            off = base + pl.multiple_of(c * R, 24)
            pltpu.sync_copy(idx_hbm.at[pl.ds(off, R)], idx_c)
            pltpu.async_copy(table_hbm.at[idx_c], rows_v, sem).wait()
            g0 = off // 6
            for k_ in range(ngroups):
                pltpu.sync_copy(
                    rows_v.at[pl.ds(6 * k_, 6)],
                    out_hbm.at[pl.ds(8 * (g0 + k_), 6)],
                )
            return carry

        lax.fori_loop(0, nchunks, body, 0)

    return k


def _make_gather_flat(Dp: int, R: int):
    nchunks = BPW // R
    mesh = plsc.VectorSubcoreMesh(core_axis_name="c", subcore_axis_name="s")

    @functools.partial(
        pl.kernel,
        out_type=jax.ShapeDtypeStruct((B, Dp), jnp.float32),
        mesh=mesh,
        scratch_types=[
            pltpu.VMEM((R,), jnp.int32),
            pltpu.VMEM((R, Dp), jnp.float32),
            pltpu.SemaphoreType.DMA,
        ],
        compiler_params=pltpu.CompilerParams(use_tc_tiling_on_sc=False),
    )
    def k(idx_hbm, table_hbm, out_hbm, idx_c, rows_v, sem):
        wid = lax.axis_index("s") * NC + lax.axis_index("c")
        base = wid * BPW

        def body(c, carry):
            off = base + pl.multiple_of(c * R, 8)
            pltpu.sync_copy(idx_hbm.at[pl.ds(off, R)], idx_c)
            pltpu.async_copy(table_hbm.at[idx_c], rows_v, sem).wait()
            pltpu.sync_copy(rows_v, out_hbm.at[pl.ds(off, R)])
            return carry

        lax.fori_loop(0, nchunks, body, 0)

    return k


def _pad8(d: int) -> int:
    return (d + 7) // 8 * 8


_CHUNK = {2047: 24, 1023: 48, 511: 96}
_KERNELS = {D: _make_gather_grouped(_pad8(D), R) for D, R in _CHUNK.items()}
_TERA = _make_gather_flat(_pad8(19), 128)


def kernel(x, species, abilities, items, movesets, teratypes):
    idx = x.reshape(-1).astype(jnp.int32)
    outs = []
    for table in (species, abilities, items, movesets):
        D = table.shape[1]
        Dp = _pad8(D)
        tp = table if Dp == D else jnp.pad(table, ((0, 0), (0, Dp - D)))
        out = _KERNELS[D](idx, tp).reshape(4096, 8, Dp)
        outs.append(lax.slice(out, (0, 0, 0), (4096, 6, D)))
    tp = jnp.pad(teratypes, ((0, 0), (0, _pad8(19) - 19)))
    out = _TERA(idx, tp)
    outs.append(out[:, :19].reshape(x.shape[0], x.shape[1], 19))
    return (outs[0], outs[1], outs[2], outs[3], outs[4])
